# baseline (speedup 1.0000x reference)
import functools

import jax
import jax.numpy as jnp
from jax import lax
from jax.experimental import pallas as pl
from jax.experimental.pallas import tpu as pltpu
from jax.experimental.pallas import tpu_sc as plsc

B = 16384
L = 50
NINP = 64
TOT = B * L
NW = 32
PER_W = TOT // NW
SLAB = 400
NSLAB = PER_W // SLAB
NBUF = 4

_mesh = plsc.VectorSubcoreMesh(core_axis_name="c", subcore_axis_name="s")


@functools.partial(
    pl.kernel,
    mesh=_mesh,
    compiler_params=pltpu.CompilerParams(use_tc_tiling_on_sc=False),
    out_type=jax.ShapeDtypeStruct((TOT, NINP), jnp.float32),
    scratch_types=[
        pltpu.VMEM((PER_W,), jnp.int32),
        pltpu.VMEM((NBUF, SLAB, NINP), jnp.float32),
        pltpu.SemaphoreType.DMA,
        pltpu.SemaphoreType.DMA,
    ],
)
def _emb_lookup(idx_hbm, table_hbm, out_hbm, idx_v, bufs, gsem, ssem):
    wid = lax.axis_index("s") * 2 + lax.axis_index("c")
    base = wid * PER_W
    pltpu.sync_copy(idx_hbm.at[pl.ds(base, PER_W)], idx_v)

    def gather(s, b):
        pltpu.async_copy(
            table_hbm.at[idx_v.at[pl.ds(s * SLAB, SLAB)]], bufs.at[b], gsem
        )

    def scatter_copy(s, b):
        return pltpu.make_async_copy(
            bufs.at[b], out_hbm.at[pl.ds(base + s * SLAB, SLAB)], ssem
        )

    gather(0, 0)
    gather(1, 1)

    def block(h, _):
        for j in range(NBUF):
            g = h * NBUF + j
            pltpu.make_async_copy(
                table_hbm.at[idx_v.at[pl.ds(g * SLAB, SLAB)]],
                bufs.at[j],
                gsem,
            ).wait()
            scatter_copy(g, j).start()

            @pl.when(g >= 2)
            def _():
                scatter_copy(g - 2, (j + 2) % NBUF).wait()

            @pl.when(g + 2 < NSLAB)
            def _():
                gather(g + 2, (j + 2) % NBUF)
        return 0

    lax.fori_loop(0, NSLAB // NBUF, block, 0)
    scatter_copy(NSLAB - 2, (NSLAB - 2) % NBUF).wait()
    scatter_copy(NSLAB - 1, (NSLAB - 1) % NBUF).wait()


def kernel(input, weight):
    idx = input.reshape(TOT)
    out = _emb_lookup(idx, weight)
    return out.reshape(B, L, NINP)
